# Initial kernel scaffold; baseline (speedup 1.0000x reference)
import jax
import jax.numpy as jnp
from jax.experimental import pallas as pl


def kernel(time, y, u, z0, A, Bm, K, b, C, c, Q, R, P0):
    raise NotImplementedError("write your pallas kernel here")


# speedup vs baseline: 1.2723x; 1.2723x over previous
import jax
import jax.numpy as jnp
from jax.experimental import pallas as pl
from jax.experimental.pallas import tpu as pltpu


def _rollout_kernel(u_ref, y_ref, z0_ref, A_ref, Wuy_ref, b_ref, C_ref, c_ref,
                    Q_ref, R_ref, P0_ref,
                    ymu_ref, ycov_ref,
                    z_scr, P_scr):
    tb = pl.program_id(1)
    TT, Bb, p = ymu_ref.shape
    m = u_ref.shape[-1]

    @pl.when(tb == 0)
    def _():
        z_scr[...] = z0_ref[...]
        P_scr[...] = P0_ref[...]

    A = A_ref[...]
    C = C_ref[...]
    Q = Q_ref[...]
    R = R_ref[...]
    AT = A.T
    CT = C.T
    c_b = jnp.broadcast_to(c_ref[...], (Bb, p))

    uy = jnp.concatenate(
        [u_ref[...].reshape(TT * Bb, m), y_ref[...].reshape(TT * Bb, p)],
        axis=1)
    D = jnp.dot(uy, Wuy_ref[...],
                preferred_element_type=jnp.float32) + b_ref[...]

    z = z_scr[...]
    P = P_scr[...]

    for t in range(TT):
        z = jnp.tanh(jnp.dot(z, A, preferred_element_type=jnp.float32)
                     + D[t * Bb:(t + 1) * Bb, :])
        ymu_ref[t] = jnp.dot(z, C, preferred_element_type=jnp.float32) + c_b

        PA = jnp.dot(P, A, preferred_element_type=jnp.float32)
        P = jnp.dot(AT, PA, preferred_element_type=jnp.float32) + Q
        PC = jnp.dot(P, C, preferred_element_type=jnp.float32)
        y_cov = jnp.dot(CT, PC, preferred_element_type=jnp.float32) + R

        ycov_ref[t] = jnp.broadcast_to(y_cov[None], (Bb, p, p))

    z_scr[...] = z
    P_scr[...] = P


def kernel(time, y, u, z0, A, Bm, K, b, C, c, Q, R, P0):
    T, B, p = y.shape
    m = u.shape[-1]
    n = z0.shape[-1]

    TT = T if T <= 8 else 8
    n_blk = pl.cdiv(T, TT)
    T_pad = n_blk * TT
    if T_pad != T:
        y = jnp.concatenate([y, jnp.zeros((T_pad - T, B, p), y.dtype)], axis=0)
        u = jnp.concatenate([u, jnp.zeros((T_pad - T, B, m), u.dtype)], axis=0)
    y = y.astype(jnp.float32)
    u = u.astype(jnp.float32)

    NB = 2 if B % 2 == 0 else 1
    Bb = B // NB

    Wuy = jnp.concatenate([Bm, K], axis=0)

    const2 = lambda bb, tb: (0, 0)

    out_shapes = (
        jax.ShapeDtypeStruct((T_pad, B, p), jnp.float32),
        jax.ShapeDtypeStruct((T_pad, B, p, p), jnp.float32),
    )

    y_mu, y_cov = pl.pallas_call(
        _rollout_kernel,
        out_shape=out_shapes,
        grid=(NB, n_blk),
        in_specs=[
            pl.BlockSpec((TT, Bb, m), lambda bb, tb: (tb, bb, 0)),
            pl.BlockSpec((TT, Bb, p), lambda bb, tb: (tb, bb, 0)),
            pl.BlockSpec((Bb, n), lambda bb, tb: (bb, 0)),
            pl.BlockSpec((n, n), const2),
            pl.BlockSpec((m + p, n), const2),
            pl.BlockSpec((1, n), const2),
            pl.BlockSpec((n, p), const2),
            pl.BlockSpec((1, p), const2),
            pl.BlockSpec((n, n), const2),
            pl.BlockSpec((p, p), const2),
            pl.BlockSpec((n, n), const2),
        ],
        out_specs=[
            pl.BlockSpec((TT, Bb, p), lambda bb, tb: (tb, bb, 0)),
            pl.BlockSpec((TT, Bb, p, p), lambda bb, tb: (tb, bb, 0, 0)),
        ],
        scratch_shapes=[
            pltpu.VMEM((Bb, n), jnp.float32),
            pltpu.VMEM((n, n), jnp.float32),
        ],
        compiler_params=pltpu.CompilerParams(
            dimension_semantics=("parallel", "arbitrary"),
            vmem_limit_bytes=64 * 1024 * 1024,
        ),
    )(u, y, z0, A, Wuy, b, C, c, Q, R, P0)

    return y_mu[:T], y_cov[:T]


# speedup vs baseline: 1.2779x; 1.0044x over previous
import jax
import jax.numpy as jnp
from jax.experimental import pallas as pl
from jax.experimental.pallas import tpu as pltpu


def _rollout_kernel(u_ref, y_ref, z0_ref, A_ref, Bm_ref, K_ref, b_ref, C_ref,
                    c_ref, Q_ref, R_ref, P0_ref,
                    ymu_ref, ycov_ref,
                    z_scr, P_scr):
    tb = pl.program_id(1)
    TT, Bb, p = ymu_ref.shape
    m = u_ref.shape[-1]

    @pl.when(tb == 0)
    def _():
        z_scr[...] = z0_ref[...]
        P_scr[...] = P0_ref[...]

    A = A_ref[...]
    C = C_ref[...]
    Q = Q_ref[...]
    R = R_ref[...]
    AT = A.T
    CT = C.T
    c_b = jnp.broadcast_to(c_ref[...], (Bb, p))

    uy = jnp.concatenate(
        [u_ref[...].reshape(TT * Bb, m), y_ref[...].reshape(TT * Bb, p)],
        axis=1)
    Wuy = jnp.concatenate([Bm_ref[...], K_ref[...]], axis=0)
    D = jnp.dot(uy, Wuy,
                preferred_element_type=jnp.float32) + b_ref[...]

    z = z_scr[...]
    P = P_scr[...]

    for t in range(TT):
        z = jnp.tanh(jnp.dot(z, A, preferred_element_type=jnp.float32)
                     + D[t * Bb:(t + 1) * Bb, :])
        ymu_ref[t] = jnp.dot(z, C, preferred_element_type=jnp.float32) + c_b

        PA = jnp.dot(P, A, preferred_element_type=jnp.float32)
        P = jnp.dot(AT, PA, preferred_element_type=jnp.float32) + Q
        PC = jnp.dot(P, C, preferred_element_type=jnp.float32)
        y_cov = jnp.dot(CT, PC, preferred_element_type=jnp.float32) + R

        ycov_ref[t] = jnp.broadcast_to(y_cov[None], (Bb, p, p))

    z_scr[...] = z
    P_scr[...] = P


def kernel(time, y, u, z0, A, Bm, K, b, C, c, Q, R, P0):
    T, B, p = y.shape
    m = u.shape[-1]
    n = z0.shape[-1]

    TT = T if T <= 8 else 8
    n_blk = pl.cdiv(T, TT)
    T_pad = n_blk * TT
    if T_pad != T:
        y = jnp.concatenate([y, jnp.zeros((T_pad - T, B, p), y.dtype)], axis=0)
        u = jnp.concatenate([u, jnp.zeros((T_pad - T, B, m), u.dtype)], axis=0)
    y = y.astype(jnp.float32)
    u = u.astype(jnp.float32)

    NB = 2 if B % 2 == 0 else 1
    Bb = B // NB

    const2 = lambda bb, tb: (0, 0)

    out_shapes = (
        jax.ShapeDtypeStruct((T_pad, B, p), jnp.float32),
        jax.ShapeDtypeStruct((T_pad, B, p, p), jnp.float32),
    )

    y_mu, y_cov = pl.pallas_call(
        _rollout_kernel,
        out_shape=out_shapes,
        grid=(NB, n_blk),
        in_specs=[
            pl.BlockSpec((TT, Bb, m), lambda bb, tb: (tb, bb, 0)),
            pl.BlockSpec((TT, Bb, p), lambda bb, tb: (tb, bb, 0)),
            pl.BlockSpec((Bb, n), lambda bb, tb: (bb, 0)),
            pl.BlockSpec((n, n), const2),
            pl.BlockSpec((m, n), const2),
            pl.BlockSpec((p, n), const2),
            pl.BlockSpec((1, n), const2),
            pl.BlockSpec((n, p), const2),
            pl.BlockSpec((1, p), const2),
            pl.BlockSpec((n, n), const2),
            pl.BlockSpec((p, p), const2),
            pl.BlockSpec((n, n), const2),
        ],
        out_specs=[
            pl.BlockSpec((TT, Bb, p), lambda bb, tb: (tb, bb, 0)),
            pl.BlockSpec((TT, Bb, p, p), lambda bb, tb: (tb, bb, 0, 0)),
        ],
        scratch_shapes=[
            pltpu.VMEM((Bb, n), jnp.float32),
            pltpu.VMEM((n, n), jnp.float32),
        ],
        compiler_params=pltpu.CompilerParams(
            dimension_semantics=("parallel", "arbitrary"),
            vmem_limit_bytes=64 * 1024 * 1024,
        ),
    )(u, y, z0, A, Bm, K, b, C, c, Q, R, P0)

    return y_mu[:T], y_cov[:T]
